# baseline (speedup 1.0000x reference)
import functools

import jax
import jax.numpy as jnp
from jax import lax
from jax.experimental import pallas as pl
from jax.experimental.pallas import tpu as pltpu
from jax.experimental.pallas import tpu_sc as plsc

N = 10000
E = 320000
D = 128
D_OUT = 40

K = 128
NCH = E // K
NW = 32
ITERS = (NCH + NW - 1) // NW
RPT = 632
NPAD = 10240
DPT = NPAD // 16

_sc_mesh = plsc.VectorSubcoreMesh(core_axis_name="c", subcore_axis_name="s")


NBD = 4
ROUNDS_D = 20


def _deg_body(edge_hbm, out_hbm, idx_v, ones_v, zb_v, deg_sh, *sems):
    sem_i = sems[0:2 * NBD]
    sem_s = sems[2 * NBD:3 * NBD]
    c = lax.axis_index("c")
    s = lax.axis_index("s")
    w = s * 2 + c
    for j in range(K // 16):
        ones_v[pl.ds(j * 16, 16)] = jnp.full((16,), 1.0, jnp.float32)
    for j in range(DPT // 16):
        zb_v[pl.ds(j * 16, 16)] = jnp.zeros((16,), jnp.float32)
    pltpu.sync_copy(zb_v, deg_sh.at[pl.ds(s * DPT, DPT)])
    plsc.subcore_barrier()

    def idx_desc(i, slot):
        ch = i * NW + w
        return pltpu.make_async_copy(edge_hbm.at[1, pl.ds(ch * K, K)],
                                     idx_v.at[slot], sem_i[slot])

    def scatter_wait(slot, b):
        pltpu.make_async_copy(ones_v, deg_sh.at[idx_v.at[slot]],
                              sem_s[b]).wait()

    for b in range(NBD):
        idx_desc(b, b).start()

    def body2(r2, carry):
        for rr in range(2):
            r = r2 * 2 + rr
            for b in range(NBD):
                slot = rr * NBD + b
                i = r * NBD + b
                ch = i * NW + w

                @pl.when((r > 0) & (ch - NBD * NW < NCH))
                def _(slot=slot, b=b):
                    scatter_wait((1 - rr) * NBD + b, b)

                @pl.when(ch < NCH)
                def _(i=i, slot=slot, b=b):
                    idx_desc(i, slot).wait()
                    pltpu.async_copy(ones_v, deg_sh.at[idx_v.at[slot]],
                                     sem_s[b], add=True)

                @pl.when((i + NBD) * NW + w < NCH)
                def _(i=i, b=b):
                    idx_desc(i + NBD, (1 - rr) * NBD + b).start()

        return carry

    lax.fori_loop(0, ROUNDS_D // 2, body2, 0)
    for b in range(NBD):
        i = (ROUNDS_D - 1) * NBD + b

        @pl.when(i * NW + w < NCH)
        def _(b=b):
            scatter_wait(((ROUNDS_D - 1) % 2) * NBD + b, b)

    plsc.subcore_barrier()
    pltpu.sync_copy(deg_sh.at[pl.ds(s * DPT, DPT)],
                    out_hbm.at[c, pl.ds(s * DPT, DPT)])


_deg_kernel = pl.kernel(
    _deg_body,
    out_type=jax.ShapeDtypeStruct((2, NPAD), jnp.float32),
    mesh=_sc_mesh,
    scratch_types=[
        pltpu.VMEM((2 * NBD, K), jnp.int32),
        pltpu.VMEM((K,), jnp.float32),
        pltpu.VMEM((DPT,), jnp.float32),
        pltpu.VMEM_SHARED((NPAD,), jnp.float32),
    ] + [pltpu.SemaphoreType.DMA] * (3 * NBD),
)


NB = 3
ROUNDS = 28


def _scatter_body(t_hbm, edge_hbm, out0_hbm, out1_hbm,
                  idx_v, rows_v, acc_sh, *sems):
    sem_i = sems[0:2 * NB]
    sem_g = sems[2 * NB:3 * NB]
    sem_s = sems[3 * NB:4 * NB]
    c = lax.axis_index("c")
    s = lax.axis_index("s")
    w = s * 2 + c
    base = jnp.minimum(s * RPT, N - RPT)
    pltpu.sync_copy(t_hbm.at[pl.ds(base, RPT)],
                    acc_sh.at[pl.ds(base, RPT)])
    plsc.subcore_barrier()

    def idx_desc(i, slot):
        ch = i * NW + w
        return pltpu.make_async_copy(edge_hbm.at[:, pl.ds(ch * K, K)],
                                     idx_v.at[slot], sem_i[slot])

    def gather_desc(slot, b):
        return pltpu.make_async_copy(t_hbm.at[idx_v.at[slot, 0]],
                                     rows_v.at[b], sem_g[b])

    def scatter_wait(slot, b):
        pltpu.make_async_copy(rows_v.at[b], acc_sh.at[idx_v.at[slot, 1]],
                              sem_s[b]).wait()

    for b in range(NB):
        idx_desc(b, b).start()

    def body2(r2, carry):
        for rr in range(2):
            r = r2 * 2 + rr
            for b in range(NB):
                slot = rr * NB + b
                i = r * NB + b
                ch = i * NW + w

                @pl.when((r > 0) & (ch - NB * NW < NCH))
                def _(slot=slot, b=b):
                    scatter_wait((1 - rr) * NB + b, b)

                @pl.when(ch < NCH)
                def _(i=i, slot=slot, b=b):
                    idx_desc(i, slot).wait()
                    gather_desc(slot, b).start()

                @pl.when((i + NB) * NW + w < NCH)
                def _(i=i, b=b):
                    idx_desc(i + NB, (1 - rr) * NB + b).start()

            for b in range(NB):
                slot = rr * NB + b
                i = r * NB + b
                ch = i * NW + w

                @pl.when(ch < NCH)
                def _(slot=slot, b=b):
                    gather_desc(slot, b).wait()
                    pltpu.async_copy(rows_v.at[b],
                                     acc_sh.at[idx_v.at[slot, 1]],
                                     sem_s[b], add=True)

        return carry

    lax.fori_loop(0, ROUNDS // 2, body2, 0)
    for b in range(NB):
        i = (ROUNDS - 1) * NB + b

        @pl.when(i * NW + w < NCH)
        def _(b=b):
            scatter_wait(((ROUNDS - 1) % 2) * NB + b, b)

    plsc.subcore_barrier()

    @pl.when(c == 0)
    def _():
        pltpu.sync_copy(acc_sh.at[pl.ds(base, RPT)],
                        out0_hbm.at[pl.ds(base, RPT)])

    @pl.when(c == 1)
    def _():
        pltpu.sync_copy(acc_sh.at[pl.ds(base, RPT)],
                        out1_hbm.at[pl.ds(base, RPT)])


_scatter_kernel = pl.kernel(
    _scatter_body,
    out_type=[jax.ShapeDtypeStruct((N, D), jnp.float32),
              jax.ShapeDtypeStruct((N, D), jnp.float32)],
    mesh=_sc_mesh,
    scratch_types=[
        pltpu.VMEM((2 * NB, 2, K), jnp.int32),
        pltpu.VMEM((NB, K, D), jnp.float32),
        pltpu.VMEM_SHARED((N, D), jnp.float32),
    ] + [pltpu.SemaphoreType.DMA] * (4 * NB),
)


BLK = 1000
GRID = N // BLK


def _dinv_col(d0_ref, d1_ref):
    return lax.rsqrt(d0_ref[...] + d1_ref[...] + 1.0)


def _prep1_body(x_ref, w_ref, d0_ref, d1_ref, t_ref, dc_ref):
    dinv = _dinv_col(d0_ref, d1_ref)
    dc_ref[...] = dinv
    t_ref[...] = jnp.dot(x_ref[...], w_ref[...],
                         preferred_element_type=jnp.float32) * dinv


def _stats_body(a0_ref, a1_ref, t_ref, dc_ref, b1_ref,
                hpre_ref, st_ref):
    i = pl.program_id(0)
    dinv = dc_ref[...]
    h = dinv * (a0_ref[...] + a1_ref[...] - t_ref[...]) + b1_ref[...]
    hpre_ref[...] = h

    @pl.when(i == 0)
    def _():
        st_ref[...] = jnp.zeros_like(st_ref)

    st_ref[0:1, :] += jnp.sum(h, axis=0, keepdims=True)
    st_ref[1:2, :] += jnp.sum(h * h, axis=0, keepdims=True)


def _mid_body(hpre_ref, st_ref, g_ref, b_ref, w2_ref, dc_ref, t2_ref):
    mean = st_ref[0:1, :] * (1.0 / N)
    var = st_ref[1:2, :] * (1.0 / N) - mean * mean
    xn = (hpre_ref[...] - mean) * lax.rsqrt(var + 1e-5) * g_ref[...] + b_ref[...]
    h1 = jnp.maximum(xn, 0.0)
    t2_ref[...] = jnp.dot(h1, w2_ref[...],
                          preferred_element_type=jnp.float32) * dc_ref[...]


def _head_body(a0_ref, a1_ref, t2_ref, dc_ref, b2_ref,
               wc_ref, bc_ref, ws_ref, bs_ref, wh_ref, bh_ref, we_ref, be_ref,
               main_ref, sim_ref, homo_ref, ent_ref):
    h2 = (dc_ref[...]
          * (a0_ref[...] + a1_ref[...] - t2_ref[...]) + b2_ref[...])

    def mm(w_ref):
        return jnp.dot(h2, w_ref[...], preferred_element_type=jnp.float32)

    zc = mm(wc_ref) + bc_ref[...]
    zc = zc - jnp.max(zc, axis=1, keepdims=True)
    main_ref[...] = zc - jnp.log(jnp.sum(jnp.exp(zc), axis=1, keepdims=True))

    zs = mm(ws_ref) + bs_ref[...]
    zs = zs - jnp.max(zs, axis=1, keepdims=True)
    ezs = jnp.exp(zs)
    sim_ref[...] = ezs / jnp.sum(ezs, axis=1, keepdims=True)

    zh = mm(wh_ref) + bh_ref[...]
    homo_ref[...] = 1.0 / (1.0 + jnp.exp(-zh))

    ze = mm(we_ref) + be_ref[...]
    ent_ref[...] = 1.0 / (1.0 + jnp.exp(-ze))


def _row_spec(width):
    return pl.BlockSpec((BLK, width), lambda i: (i, 0))


def _fold_spec():
    return pl.BlockSpec((BLK, 1), lambda i: (i, 0))


def _full_spec(r, c):
    return pl.BlockSpec((r, c), lambda i: (0, 0))


@jax.jit
def kernel(x, edge_index, W1, b1, W2, b2, bn_g, bn_b,
           Wc, bc, Ws, bs, Wh, bh, We, be):
    degp = _deg_kernel(edge_index)
    d0 = degp[0, :N].reshape(N, 1)
    d1 = degp[1, :N].reshape(N, 1)

    t1, dinvc = pl.pallas_call(
        _prep1_body,
        grid=(GRID,),
        in_specs=[_row_spec(D), _full_spec(D, D), _fold_spec(), _fold_spec()],
        out_specs=[_row_spec(D), _fold_spec()],
        out_shape=[jax.ShapeDtypeStruct((N, D), jnp.float32),
                   jax.ShapeDtypeStruct((N, 1), jnp.float32)],
    )(x, W1, d0, d1)

    a1_0, a1_1 = _scatter_kernel(t1, edge_index)

    hpre, stats = pl.pallas_call(
        _stats_body,
        grid=(GRID,),
        in_specs=[_row_spec(D), _row_spec(D), _row_spec(D),
                  _fold_spec(), _full_spec(1, D)],
        out_specs=[_row_spec(D), _full_spec(2, D)],
        out_shape=[jax.ShapeDtypeStruct((N, D), jnp.float32),
                   jax.ShapeDtypeStruct((2, D), jnp.float32)],
    )(a1_0, a1_1, t1, dinvc, b1.reshape(1, D))

    t2 = pl.pallas_call(
        _mid_body,
        grid=(GRID,),
        in_specs=[_row_spec(D), _full_spec(2, D), _full_spec(1, D),
                  _full_spec(1, D), _full_spec(D, D), _fold_spec()],
        out_specs=_row_spec(D),
        out_shape=jax.ShapeDtypeStruct((N, D), jnp.float32),
    )(hpre, stats, bn_g.reshape(1, D), bn_b.reshape(1, D), W2, dinvc)

    a2_0, a2_1 = _scatter_kernel(t2, edge_index)

    main, sim, homo, ent = pl.pallas_call(
        _head_body,
        grid=(GRID,),
        in_specs=[_row_spec(D), _row_spec(D), _row_spec(D),
                  _fold_spec(), _full_spec(1, D),
                  _full_spec(D, D_OUT), _full_spec(1, D_OUT),
                  _full_spec(D, D_OUT), _full_spec(1, D_OUT),
                  _full_spec(D, 1), _full_spec(1, 1),
                  _full_spec(D, 1), _full_spec(1, 1)],
        out_specs=[_row_spec(D_OUT), _row_spec(D_OUT),
                   _row_spec(1), _row_spec(1)],
        out_shape=[jax.ShapeDtypeStruct((N, D_OUT), jnp.float32),
                   jax.ShapeDtypeStruct((N, D_OUT), jnp.float32),
                   jax.ShapeDtypeStruct((N, 1), jnp.float32),
                   jax.ShapeDtypeStruct((N, 1), jnp.float32)],
    )(a2_0, a2_1, t2, dinvc, b2.reshape(1, D),
      Wc, bc.reshape(1, D_OUT), Ws, bs.reshape(1, D_OUT),
      Wh, bh.reshape(1, 1), We, be.reshape(1, 1))

    return main, sim, homo.reshape(N), ent.reshape(N)


# speedup vs baseline: 1.0217x; 1.0217x over previous
import functools

import jax
import jax.numpy as jnp
from jax import lax
from jax.experimental import pallas as pl
from jax.experimental.pallas import tpu as pltpu
from jax.experimental.pallas import tpu_sc as plsc

N = 10000
E = 320000
D = 128
D_OUT = 40

K = 128
NCH = E // K
NW = 32
ITERS = (NCH + NW - 1) // NW
RPT = 632
NPAD = 10240
DPT = NPAD // 16

_sc_mesh = plsc.VectorSubcoreMesh(core_axis_name="c", subcore_axis_name="s")


NBD = 4
ROUNDS_D = 20


def _deg_body(edge_hbm, out_hbm, idx_v, ones_v, zb_v, deg_sh, *sems):
    sem_i = sems[0:2 * NBD]
    sem_s = sems[2 * NBD:3 * NBD]
    c = lax.axis_index("c")
    s = lax.axis_index("s")
    w = s * 2 + c
    for j in range(K // 16):
        ones_v[pl.ds(j * 16, 16)] = jnp.full((16,), 1.0, jnp.float32)
    for j in range(DPT // 16):
        zb_v[pl.ds(j * 16, 16)] = jnp.zeros((16,), jnp.float32)
    pltpu.sync_copy(zb_v, deg_sh.at[pl.ds(s * DPT, DPT)])
    plsc.subcore_barrier()

    def idx_desc(i, slot):
        ch = i * NW + w
        return pltpu.make_async_copy(edge_hbm.at[1, pl.ds(ch * K, K)],
                                     idx_v.at[slot], sem_i[slot])

    def scatter_wait(slot, b):
        pltpu.make_async_copy(ones_v, deg_sh.at[idx_v.at[slot]],
                              sem_s[b]).wait()

    for b in range(NBD):
        idx_desc(b, b).start()

    def body2(r2, carry):
        for rr in range(2):
            r = r2 * 2 + rr
            for b in range(NBD):
                slot = rr * NBD + b
                i = r * NBD + b
                ch = i * NW + w

                @pl.when((r > 0) & (ch - NBD * NW < NCH))
                def _(slot=slot, b=b):
                    scatter_wait((1 - rr) * NBD + b, b)

                @pl.when(ch < NCH)
                def _(i=i, slot=slot, b=b):
                    idx_desc(i, slot).wait()
                    pltpu.async_copy(ones_v, deg_sh.at[idx_v.at[slot]],
                                     sem_s[b], add=True)

                @pl.when((i + NBD) * NW + w < NCH)
                def _(i=i, b=b):
                    idx_desc(i + NBD, (1 - rr) * NBD + b).start()

        return carry

    lax.fori_loop(0, ROUNDS_D // 2, body2, 0)
    for b in range(NBD):
        i = (ROUNDS_D - 1) * NBD + b

        @pl.when(i * NW + w < NCH)
        def _(b=b):
            scatter_wait(((ROUNDS_D - 1) % 2) * NBD + b, b)

    plsc.subcore_barrier()
    pltpu.sync_copy(deg_sh.at[pl.ds(s * DPT, DPT)],
                    out_hbm.at[c, pl.ds(s * DPT, DPT)])


_deg_kernel = pl.kernel(
    _deg_body,
    out_type=jax.ShapeDtypeStruct((2, NPAD), jnp.float32),
    mesh=_sc_mesh,
    scratch_types=[
        pltpu.VMEM((2 * NBD, K), jnp.int32),
        pltpu.VMEM((K,), jnp.float32),
        pltpu.VMEM((DPT,), jnp.float32),
        pltpu.VMEM_SHARED((NPAD,), jnp.float32),
    ] + [pltpu.SemaphoreType.DMA] * (3 * NBD),
)


NB = 3
ROUNDS = 28


def _scatter_body(t_hbm, edge_hbm, out0_hbm, out1_hbm,
                  idx_v, rows_v, acc_sh, *sems):
    sem_i = sems[0:2 * NB]
    sem_g = sems[2 * NB:3 * NB]
    sem_s = sems[3 * NB:4 * NB]
    c = lax.axis_index("c")
    s = lax.axis_index("s")
    w = s * 2 + c
    base = jnp.minimum(s * RPT, N - RPT)
    pltpu.sync_copy(t_hbm.at[pl.ds(base, RPT)],
                    acc_sh.at[pl.ds(base, RPT)])
    plsc.subcore_barrier()

    def idx_desc(i, slot):
        ch = i * NW + w
        return pltpu.make_async_copy(edge_hbm.at[:, pl.ds(ch * K, K)],
                                     idx_v.at[slot], sem_i[slot])

    def gather_desc(slot, b):
        return pltpu.make_async_copy(t_hbm.at[idx_v.at[slot, 0]],
                                     rows_v.at[b], sem_g[b])

    def scatter_wait(slot, b):
        pltpu.make_async_copy(rows_v.at[b], acc_sh.at[idx_v.at[slot, 1]],
                              sem_s[b]).wait()

    for b in range(NB):
        idx_desc(b, b).start()

    def body2(r2, carry):
        for rr in range(2):
            r = r2 * 2 + rr
            for b in range(NB):
                slot = rr * NB + b
                i = r * NB + b
                ch = i * NW + w

                @pl.when((r > 0) & (ch - NB * NW < NCH))
                def _(slot=slot, b=b):
                    scatter_wait((1 - rr) * NB + b, b)

                @pl.when(ch < NCH)
                def _(i=i, slot=slot, b=b):
                    idx_desc(i, slot).wait()
                    gather_desc(slot, b).start()

                @pl.when((i + NB) * NW + w < NCH)
                def _(i=i, b=b):
                    idx_desc(i + NB, (1 - rr) * NB + b).start()

            for b in range(NB):
                slot = rr * NB + b
                i = r * NB + b
                ch = i * NW + w

                @pl.when(ch < NCH)
                def _(slot=slot, b=b):
                    gather_desc(slot, b).wait()
                    pltpu.async_copy(rows_v.at[b],
                                     acc_sh.at[idx_v.at[slot, 1]],
                                     sem_s[b], add=True)

        return carry

    lax.fori_loop(0, ROUNDS // 2, body2, 0)
    for b in range(NB):
        i = (ROUNDS - 1) * NB + b

        @pl.when(i * NW + w < NCH)
        def _(b=b):
            scatter_wait(((ROUNDS - 1) % 2) * NB + b, b)

    plsc.subcore_barrier()

    @pl.when(c == 0)
    def _():
        pltpu.sync_copy(acc_sh.at[pl.ds(base, RPT)],
                        out0_hbm.at[pl.ds(base, RPT)])

    @pl.when(c == 1)
    def _():
        pltpu.sync_copy(acc_sh.at[pl.ds(base, RPT)],
                        out1_hbm.at[pl.ds(base, RPT)])


_scatter_kernel = pl.kernel(
    _scatter_body,
    out_type=[jax.ShapeDtypeStruct((N, D), jnp.float32),
              jax.ShapeDtypeStruct((N, D), jnp.float32)],
    mesh=_sc_mesh,
    scratch_types=[
        pltpu.VMEM((2 * NB, 2, K), jnp.int32),
        pltpu.VMEM((NB, K, D), jnp.float32),
        pltpu.VMEM_SHARED((N, D), jnp.float32),
    ] + [pltpu.SemaphoreType.DMA] * (4 * NB),
)


BLK = 2000
GRID = N // BLK


def _dinv_col(d0_ref, d1_ref):
    return lax.rsqrt(d0_ref[...] + d1_ref[...] + 1.0)


def _prep1_body(x_ref, w_ref, d0_ref, d1_ref, t_ref, dc_ref):
    dinv = _dinv_col(d0_ref, d1_ref)
    dc_ref[...] = dinv
    t_ref[...] = jnp.dot(x_ref[...], w_ref[...],
                         preferred_element_type=jnp.float32) * dinv


def _stats_body(a0_ref, a1_ref, t_ref, dc_ref, b1_ref,
                hpre_ref, st_ref):
    i = pl.program_id(0)
    dinv = dc_ref[...]
    h = dinv * (a0_ref[...] + a1_ref[...] - t_ref[...]) + b1_ref[...]
    hpre_ref[...] = h

    @pl.when(i == 0)
    def _():
        st_ref[...] = jnp.zeros_like(st_ref)

    st_ref[0:1, :] += jnp.sum(h, axis=0, keepdims=True)
    st_ref[1:2, :] += jnp.sum(h * h, axis=0, keepdims=True)


def _mid_body(hpre_ref, st_ref, g_ref, b_ref, w2_ref, dc_ref, t2_ref):
    mean = st_ref[0:1, :] * (1.0 / N)
    var = st_ref[1:2, :] * (1.0 / N) - mean * mean
    xn = (hpre_ref[...] - mean) * lax.rsqrt(var + 1e-5) * g_ref[...] + b_ref[...]
    h1 = jnp.maximum(xn, 0.0)
    t2_ref[...] = jnp.dot(h1, w2_ref[...],
                          preferred_element_type=jnp.float32) * dc_ref[...]


def _head_body(a0_ref, a1_ref, t2_ref, dc_ref, b2_ref,
               wc_ref, bc_ref, ws_ref, bs_ref, wh_ref, bh_ref, we_ref, be_ref,
               main_ref, sim_ref, homo_ref, ent_ref):
    h2 = (dc_ref[...]
          * (a0_ref[...] + a1_ref[...] - t2_ref[...]) + b2_ref[...])

    def mm(w_ref):
        return jnp.dot(h2, w_ref[...], preferred_element_type=jnp.float32)

    zc = mm(wc_ref) + bc_ref[...]
    zc = zc - jnp.max(zc, axis=1, keepdims=True)
    main_ref[...] = zc - jnp.log(jnp.sum(jnp.exp(zc), axis=1, keepdims=True))

    zs = mm(ws_ref) + bs_ref[...]
    zs = zs - jnp.max(zs, axis=1, keepdims=True)
    ezs = jnp.exp(zs)
    sim_ref[...] = ezs / jnp.sum(ezs, axis=1, keepdims=True)

    zh = mm(wh_ref) + bh_ref[...]
    homo_ref[...] = 1.0 / (1.0 + jnp.exp(-zh))

    ze = mm(we_ref) + be_ref[...]
    ent_ref[...] = 1.0 / (1.0 + jnp.exp(-ze))


def _row_spec(width):
    return pl.BlockSpec((BLK, width), lambda i: (i, 0))


def _fold_spec():
    return pl.BlockSpec((BLK, 1), lambda i: (i, 0))


def _full_spec(r, c):
    return pl.BlockSpec((r, c), lambda i: (0, 0))


@jax.jit
def kernel(x, edge_index, W1, b1, W2, b2, bn_g, bn_b,
           Wc, bc, Ws, bs, Wh, bh, We, be):
    degp = _deg_kernel(edge_index)
    d0 = degp[0, :N].reshape(N, 1)
    d1 = degp[1, :N].reshape(N, 1)

    t1, dinvc = pl.pallas_call(
        _prep1_body,
        grid=(GRID,),
        in_specs=[_row_spec(D), _full_spec(D, D), _fold_spec(), _fold_spec()],
        out_specs=[_row_spec(D), _fold_spec()],
        out_shape=[jax.ShapeDtypeStruct((N, D), jnp.float32),
                   jax.ShapeDtypeStruct((N, 1), jnp.float32)],
    )(x, W1, d0, d1)

    a1_0, a1_1 = _scatter_kernel(t1, edge_index)

    hpre, stats = pl.pallas_call(
        _stats_body,
        grid=(GRID,),
        in_specs=[_row_spec(D), _row_spec(D), _row_spec(D),
                  _fold_spec(), _full_spec(1, D)],
        out_specs=[_row_spec(D), _full_spec(2, D)],
        out_shape=[jax.ShapeDtypeStruct((N, D), jnp.float32),
                   jax.ShapeDtypeStruct((2, D), jnp.float32)],
    )(a1_0, a1_1, t1, dinvc, b1.reshape(1, D))

    t2 = pl.pallas_call(
        _mid_body,
        grid=(GRID,),
        in_specs=[_row_spec(D), _full_spec(2, D), _full_spec(1, D),
                  _full_spec(1, D), _full_spec(D, D), _fold_spec()],
        out_specs=_row_spec(D),
        out_shape=jax.ShapeDtypeStruct((N, D), jnp.float32),
    )(hpre, stats, bn_g.reshape(1, D), bn_b.reshape(1, D), W2, dinvc)

    a2_0, a2_1 = _scatter_kernel(t2, edge_index)

    main, sim, homo, ent = pl.pallas_call(
        _head_body,
        grid=(GRID,),
        in_specs=[_row_spec(D), _row_spec(D), _row_spec(D),
                  _fold_spec(), _full_spec(1, D),
                  _full_spec(D, D_OUT), _full_spec(1, D_OUT),
                  _full_spec(D, D_OUT), _full_spec(1, D_OUT),
                  _full_spec(D, 1), _full_spec(1, 1),
                  _full_spec(D, 1), _full_spec(1, 1)],
        out_specs=[_row_spec(D_OUT), _row_spec(D_OUT),
                   _row_spec(1), _row_spec(1)],
        out_shape=[jax.ShapeDtypeStruct((N, D_OUT), jnp.float32),
                   jax.ShapeDtypeStruct((N, D_OUT), jnp.float32),
                   jax.ShapeDtypeStruct((N, 1), jnp.float32),
                   jax.ShapeDtypeStruct((N, 1), jnp.float32)],
    )(a2_0, a2_1, t2, dinvc, b2.reshape(1, D),
      Wc, bc.reshape(1, D_OUT), Ws, bs.reshape(1, D_OUT),
      Wh, bh.reshape(1, 1), We, be.reshape(1, 1))

    return main, sim, homo.reshape(N), ent.reshape(N)
